# baseline (speedup 1.0000x reference)
import functools

import jax
import jax.numpy as jnp
from jax import lax
from jax.experimental import pallas as pl
from jax.experimental.pallas import tpu as pltpu
from jax.experimental.pallas import tpu_sc as plsc

NCOMP = 10000
NSEC = 100
PER = 100
WIN = 16
IND = 64
HID = 128
ODIM = 5
NIN = 320000
NOUT = 10000

HAF = 80
HSPLIT = 80
SC_CORES = 2
SC_TILES = 16
E_TILE = NIN // SC_TILES
CHUNK = 80
NCHUNK = E_TILE // CHUNK
ROWS_T = NCOMP // SC_TILES
ZROWS = 125
CBLK = 1000
GRID = NCOMP // CBLK
F32 = jnp.float32


def _bn_stats_body(x_ref, gam_ref, bet_ref, scale_ref, shift_ref, acc_ref):
    i = pl.program_id(0)

    @pl.when(i == 0)
    def _():
        acc_ref[...] = jnp.zeros_like(acc_ref)

    x = x_ref[...]
    acc_ref[0] = acc_ref[0] + jnp.sum(x, axis=1)
    acc_ref[1] = acc_ref[1] + jnp.sum(x * x, axis=1)

    @pl.when(i == GRID - 1)
    def _():
        n = float(NCOMP)
        mean = acc_ref[0] / n
        var = acc_ref[1] / n - mean * mean
        sc = gam_ref[...] * lax.rsqrt(var + 1e-5)
        scale_ref[...] = sc
        shift_ref[...] = bet_ref[...] - mean * sc


def _bn_stats(daily, gam, bet):
    return pl.pallas_call(
        _bn_stats_body,
        grid=(GRID,),
        in_specs=[
            pl.BlockSpec((WIN, CBLK, IND), lambda i: (0, i, 0)),
            pl.BlockSpec((WIN, IND), lambda i: (0, 0)),
            pl.BlockSpec((WIN, IND), lambda i: (0, 0)),
        ],
        out_specs=[
            pl.BlockSpec((WIN, IND), lambda i: (0, 0)),
            pl.BlockSpec((WIN, IND), lambda i: (0, 0)),
        ],
        out_shape=[
            jax.ShapeDtypeStruct((WIN, IND), F32),
            jax.ShapeDtypeStruct((WIN, IND), F32),
        ],
        scratch_shapes=[pltpu.VMEM((2, WIN, IND), F32)],
    )(daily, gam, bet)


def _gru_body(x_ref, scale_ref, shift_ref, wi_ref, wh_ref, bi_ref, bh_ref,
              w1_ref, asrc_ref, adst_ref,
              seq_ref, auga_ref, augb_ref, ss_ref, sd_ref):
    x = x_ref[...]
    x = x * scale_ref[...][:, None, :] + shift_ref[...][:, None, :]
    wi = wi_ref[...]
    wh = wh_ref[...]
    bi = bi_ref[...]
    bh = bh_ref[...]
    h = jnp.zeros((CBLK, HID), F32)
    for t in range(WIN):
        gi = jnp.dot(x[t], wi, preferred_element_type=F32) + bi
        gh = jnp.dot(h, wh, preferred_element_type=F32) + bh
        r = jax.nn.sigmoid(gi[:, :HID] + gh[:, :HID])
        z = jax.nn.sigmoid(gi[:, HID:2 * HID] + gh[:, HID:2 * HID])
        n = jnp.tanh(gi[:, 2 * HID:] + r * gh[:, 2 * HID:])
        h = (1.0 - z) * n + z * h
    seq_ref[...] = h
    h1 = jnp.dot(h, w1_ref[...], preferred_element_type=F32)
    auga_ref[...] = h1[:, :HSPLIT]
    augb_ref[...] = jnp.concatenate(
        [h1[:, HSPLIT:], jnp.ones((CBLK, HAF - (HID - HSPLIT)), F32)], axis=1)
    ss_ref[...] = jnp.dot(h1, asrc_ref[...], preferred_element_type=F32)
    sd_ref[...] = jnp.dot(h1, adst_ref[...], preferred_element_type=F32)


def _gru_head(daily, scale, shift, wi, wh, bi, bh, w1, asrc, adst):
    full = lambda shape: pl.BlockSpec(shape, lambda i: tuple(0 for _ in shape))
    return pl.pallas_call(
        _gru_body,
        grid=(GRID,),
        in_specs=[
            pl.BlockSpec((WIN, CBLK, IND), lambda i: (0, i, 0)),
            full((WIN, IND)), full((WIN, IND)),
            full((IND, 3 * HID)), full((HID, 3 * HID)),
            full((1, 3 * HID)), full((1, 3 * HID)),
            full((HID, HID)), full((HID, 1)), full((HID, 1)),
        ],
        out_specs=[
            pl.BlockSpec((CBLK, HID), lambda i: (i, 0)),
            pl.BlockSpec((CBLK, HAF), lambda i: (i, 0)),
            pl.BlockSpec((CBLK, HAF), lambda i: (i, 0)),
            pl.BlockSpec((CBLK, 1), lambda i: (i, 0)),
            pl.BlockSpec((CBLK, 1), lambda i: (i, 0)),
        ],
        out_shape=[
            jax.ShapeDtypeStruct((NCOMP, HID), F32),
            jax.ShapeDtypeStruct((NCOMP, HAF), F32),
            jax.ShapeDtypeStruct((NCOMP, HAF), F32),
            jax.ShapeDtypeStruct((NCOMP, 1), F32),
            jax.ShapeDtypeStruct((NCOMP, 1), F32),
        ],
    )(daily, scale, shift, wi, wh, bi, bh, w1, asrc, adst)


NBUF = 5
RCHUNK = 50
SROUND = NCHUNK // RCHUNK


def _gat1_edges_body(src2_hbm, dst2_hbm, ssrc_hbm, sdst_hbm, aug2_hbm,
                     out_hbm, *refs):
    (ssrc_v, sdst_v, src2_v, dst2_v, exv_all, acc_sh) = refs[:6]
    bufs = refs[6:6 + NBUF]
    gsems = refs[6 + NBUF:6 + 2 * NBUF]
    ssems = refs[6 + 2 * NBUF:6 + 3 * NBUF]
    cid = lax.axis_index("c")
    sid = lax.axis_index("s")

    pltpu.sync_copy(ssrc_hbm, ssrc_v)
    pltpu.sync_copy(sdst_hbm, sdst_v)

    def z_body(i, carry):
        for j in range(HAF // 16):
            exv_all[i, pl.ds(j * 16, 16)] = jnp.zeros((16,), F32)
        return carry
    lax.fori_loop(0, RCHUNK, z_body, 0)
    row0 = sid * ROWS_T
    for z in range(ROWS_T // RCHUNK):
        pltpu.sync_copy(exv_all.at[pl.ds(0, RCHUNK)],
                        acc_sh.at[pl.ds(row0 + z * RCHUNK, RCHUNK)])
    pltpu.sync_copy(exv_all.at[pl.ds(0, ROWS_T % RCHUNK)],
                    acc_sh.at[pl.ds(row0 + ROWS_T - ROWS_T % RCHUNK,
                                    ROWS_T % RCHUNK)])
    plsc.subcore_barrier()

    rebase = cid * NCOMP

    def gather_start(b, kk):
        pltpu.async_copy(aug2_hbm.at[src2_v.at[kk]], bufs[b], gsems[b])

    def gather_wait(b, kk):
        pltpu.make_async_copy(aug2_hbm.at[src2_v.at[kk]], bufs[b],
                              gsems[b]).wait()

    def scat_start(b, kk):
        pltpu.async_copy(bufs[b], acc_sh.at[dst2_v.at[kk]], ssems[b],
                         add=True)

    def scat_wait(b, kk):
        pltpu.make_async_copy(bufs[b], acc_sh.at[dst2_v.at[kk]],
                              ssems[b]).wait()

    def weight(b, kk):
        @plsc.parallel_loop(0, CHUNK // 16)
        def w_i(i):
            exvec = exv_all[kk, pl.ds(i * 16, 16)]
            for l in range(16):
                a = exvec[l]
                row = i * 16 + l
                for j in range(HAF // 16):
                    sl = pl.ds(j * 16, 16)
                    bufs[b][row, sl] = bufs[b][row, sl] * a

    def super_round(r, carry):
        crow0 = sid * NCHUNK + r * RCHUNK
        pltpu.sync_copy(src2_hbm.at[pl.ds(crow0, RCHUNK)], src2_v)
        pltpu.sync_copy(dst2_hbm.at[pl.ds(crow0, RCHUNK)], dst2_v)

        @plsc.parallel_loop(0, RCHUNK)
        def ex_body(k):
            for i in range(CHUNK // 16):
                sl = pl.ds(i * 16, 16)
                s_ids = src2_v[k, sl]
                e = (plsc.load_gather(ssrc_v, [s_ids])
                     + plsc.load_gather(sdst_v, [dst2_v[k, sl]]))
                e = jnp.where(e >= 0.0, e, 0.2 * e)
                exv_all[k, sl] = jnp.exp(e)
                src2_v[k, sl] = s_ids + rebase

        for b in range(NBUF):
            gather_start(b, b)

        def round_body(it, c2):
            k0 = it * NBUF
            for b in range(NBUF):
                kk = k0 + b
                gather_wait(b, kk)
                weight(b, kk)
                scat_start(b, kk)
                bp = (b + 2) % NBUF

                @pl.when((kk >= 3) & (kk + 2 < RCHUNK))
                def _():
                    scat_wait(bp, kk - 3)
                    gather_start(bp, kk + 2)
            return c2
        lax.fori_loop(0, RCHUNK // NBUF, round_body, 0)
        for b in range(NBUF):
            scat_wait(b, RCHUNK - NBUF + b)
        return carry
    lax.fori_loop(0, SROUND, super_round, 0)

    plsc.subcore_barrier()
    pltpu.sync_copy(acc_sh.at[pl.ds(row0, ROWS_T)],
                    out_hbm.at[pl.ds(cid * NCOMP + row0, ROWS_T)])


@functools.lru_cache(maxsize=1)
def _gat1_edges_call():
    mesh = plsc.VectorSubcoreMesh(
        core_axis_name="c", subcore_axis_name="s",
        num_cores=SC_CORES, num_subcores=SC_TILES)
    scratch = [
        pltpu.VMEM((NCOMP,), F32),
        pltpu.VMEM((NCOMP,), F32),
        pltpu.VMEM((RCHUNK, CHUNK), jnp.int32),
        pltpu.VMEM((RCHUNK, CHUNK), jnp.int32),
        pltpu.VMEM((RCHUNK, CHUNK), F32),
        pltpu.VMEM_SHARED((NCOMP, HAF), F32),
    ]
    scratch += [pltpu.VMEM((CHUNK, HAF), F32) for _ in range(NBUF)]
    scratch += [pltpu.SemaphoreType.DMA for _ in range(2 * NBUF)]
    return pl.kernel(
        _gat1_edges_body,
        out_type=jax.ShapeDtypeStruct((SC_CORES * NCOMP, HAF), F32),
        mesh=mesh,
        scratch_types=scratch,
        compiler_params=pltpu.CompilerParams(
            use_tc_tiling_on_sc=False, needs_layout_passes=False),
    )


def _gat1_edges(src, dst, ss, sd, auga, augb):
    aug2 = jnp.concatenate([auga, augb], axis=0)
    src2 = src.reshape(NIN // CHUNK, CHUNK)
    dst2 = dst.reshape(NIN // CHUNK, CHUNK)
    return _gat1_edges_call()(src2, dst2, ss, sd, aug2)


def _intra_body(p0_ref, p1_ref, b1_ref, intra_ref, sec_ref):
    p0 = p0_ref[...]
    p1 = p1_ref[...]
    acc = jnp.concatenate([p0, p1[:, :HID - HSPLIT]], axis=1)
    den = p1[:, HID - HSPLIT:HID - HSPLIT + 1]
    intra = acc / (den + 1e-16) + b1_ref[...]
    intra_ref[...] = intra
    sec_ref[0] = jnp.max(intra.reshape(CBLK // PER, PER, HID), axis=1)


def _intra_pool(parts, b1):
    return pl.pallas_call(
        _intra_body,
        grid=(GRID,),
        in_specs=[
            pl.BlockSpec((CBLK, HAF), lambda i: (i, 0)),
            pl.BlockSpec((CBLK, HAF), lambda i: (i + GRID, 0)),
            pl.BlockSpec((1, HID), lambda i: (0, 0)),
        ],
        out_specs=[
            pl.BlockSpec((CBLK, HID), lambda i: (i, 0)),
            pl.BlockSpec((1, CBLK // PER, HID), lambda i: (i, 0, 0)),
        ],
        out_shape=[
            jax.ShapeDtypeStruct((NCOMP, HID), F32),
            jax.ShapeDtypeStruct((GRID, CBLK // PER, HID), F32),
        ],
    )(parts, parts, b1)


def _gat2_body(sec_ref, osrc_ref, odst_ref, w2_ref, a2s_ref, a2d_ref, b2_ref,
               sec2_ref):
    sec = sec_ref[...]
    h2 = jnp.dot(sec, w2_ref[...], preferred_element_type=F32)
    vs = jnp.dot(h2, a2s_ref[...], preferred_element_type=F32)
    vd = jnp.dot(h2, a2d_ref[...], preferred_element_type=F32)
    k = lax.broadcasted_iota(jnp.int32, (1, NSEC), 1)
    ohs = (osrc_ref[...] == k).astype(F32)
    ohd = (odst_ref[...] == k).astype(F32)
    e = (jnp.dot(ohs, vs, preferred_element_type=F32)
         + jnp.dot(ohd, vd, preferred_element_type=F32))
    e = jnp.where(e >= 0.0, e, 0.2 * e)
    ex = jnp.exp(e)
    wdst = ohd * ex
    wss = lax.dot_general(wdst, ohs, (((0,), (0,)), ((), ())),
                          preferred_element_type=F32)
    acc2 = jnp.dot(wss, h2, preferred_element_type=F32)
    den2 = jnp.sum(wss, axis=1, keepdims=True)
    sec2_ref[...] = acc2 / (den2 + 1e-16) + b2_ref[...]


def _gat2(sec, osrc, odst, w2, a2s, a2d, b2):
    return pl.pallas_call(
        _gat2_body,
        out_shape=jax.ShapeDtypeStruct((NSEC, HID), F32),
    )(sec, osrc, odst, w2, a2s, a2d, b2)


def _head_body(seq_ref, intra_ref, sec2_ref, fw_ref, fb_ref, lw_ref, lb_ref,
               out_ref):
    secb = sec2_ref[0]
    rep = jnp.broadcast_to(secb[:, None, :], (CBLK // PER, PER, HID))
    rep = rep.reshape(CBLK, HID)
    cat = jnp.concatenate([seq_ref[...], rep, intra_ref[...]], axis=1)
    f = jnp.dot(cat, fw_ref[...], preferred_element_type=F32) + fb_ref[...]
    f = jnp.maximum(f, 0.0)
    lo = jnp.dot(f, lw_ref[...], preferred_element_type=F32) + lb_ref[...]
    m = jnp.max(lo, axis=1, keepdims=True)
    p = jnp.exp(lo - m)
    sm = p / jnp.sum(p, axis=1, keepdims=True)
    ii = lax.broadcasted_iota(jnp.int32, (ODIM, ODIM), 0)
    jj = lax.broadcasted_iota(jnp.int32, (ODIM, ODIM), 1)
    tri = (ii <= jj).astype(F32)
    cum = jnp.dot(sm, tri, preferred_element_type=F32)
    out_ref[...] = jnp.clip(cum, 5e-8, 1.0 - 5e-8)


def _head(seq, intra, sec2, fw, fb, lw, lb):
    full = lambda shape: pl.BlockSpec(shape, lambda i: tuple(0 for _ in shape))
    return pl.pallas_call(
        _head_body,
        grid=(GRID,),
        in_specs=[
            pl.BlockSpec((CBLK, HID), lambda i: (i, 0)),
            pl.BlockSpec((CBLK, HID), lambda i: (i, 0)),
            pl.BlockSpec((1, CBLK // PER, HID), lambda i: (i, 0, 0)),
            full((3 * HID, HID)), full((1, HID)),
            full((HID, ODIM)), full((1, ODIM)),
        ],
        out_specs=[pl.BlockSpec((CBLK, ODIM), lambda i: (i, 0))],
        out_shape=[jax.ShapeDtypeStruct((NCOMP, ODIM), F32)],
    )(seq, intra, sec2, fw, fb, lw, lb)[0]


def kernel(daily_data_batch, inner_edge, outer_edge, bn_gamma, bn_beta,
           gru_Wi, gru_Wh, gru_bi, gru_bh, gat1_W, gat1_a_src, gat1_a_dst,
           gat1_b, gat2_W, gat2_a_src, gat2_a_dst, gat2_b, fusion_W,
           fusion_b, logit_W, logit_b):
    gam = bn_gamma.reshape(WIN, IND)
    bet = bn_beta.reshape(WIN, IND)
    scale, shift = _bn_stats(daily_data_batch, gam, bet)
    seq, auga, augb, ss, sd = _gru_head(
        daily_data_batch, scale, shift, gru_Wi, gru_Wh,
        gru_bi.reshape(1, 3 * HID), gru_bh.reshape(1, 3 * HID),
        gat1_W, gat1_a_src.reshape(HID, 1), gat1_a_dst.reshape(HID, 1))
    parts = _gat1_edges(inner_edge[0], inner_edge[1],
                        ss.reshape(NCOMP), sd.reshape(NCOMP), auga, augb)
    intra, sec = _intra_pool(parts, gat1_b.reshape(1, HID))
    sec = sec.reshape(NSEC, HID)
    sec2 = _gat2(sec, outer_edge[0].reshape(NOUT, 1),
                 outer_edge[1].reshape(NOUT, 1), gat2_W,
                 gat2_a_src.reshape(HID, 1), gat2_a_dst.reshape(HID, 1),
                 gat2_b.reshape(1, HID))
    return _head(seq, intra, sec2.reshape(GRID, CBLK // PER, HID),
                 fusion_W, fusion_b.reshape(1, HID),
                 logit_W, logit_b.reshape(1, ODIM))


# speedup vs baseline: 1.0605x; 1.0605x over previous
import functools

import jax
import jax.numpy as jnp
from jax import lax
from jax.experimental import pallas as pl
from jax.experimental.pallas import tpu as pltpu
from jax.experimental.pallas import tpu_sc as plsc

NCOMP = 10000
NSEC = 100
PER = 100
WIN = 16
IND = 64
HID = 128
ODIM = 5
NIN = 320000
NOUT = 10000

HAF = 80
HSPLIT = 80
SC_CORES = 2
SC_TILES = 16
E_TILE = NIN // SC_TILES
CHUNK = 80
NCHUNK = E_TILE // CHUNK
ROWS_T = NCOMP // SC_TILES
ZROWS = 125
CBLK = 1000
GRID = NCOMP // CBLK
F32 = jnp.float32


def _k1_body(x_ref, gam_ref, bet_ref, wi_ref, wh_ref, bi_ref, bh_ref,
             w1_ref, asrc_ref, adst_ref,
             seq_ref, aug_ref, ss_ref, sd_ref,
             acc_ref, scale_ref, shift_ref):
    p = pl.program_id(0)
    i = pl.program_id(1)

    @pl.when(p == 0)
    def _():
        @pl.when(i == 0)
        def _():
            acc_ref[...] = jnp.zeros_like(acc_ref)

        x = x_ref[...]
        acc_ref[0] = acc_ref[0] + jnp.sum(x, axis=1)
        acc_ref[1] = acc_ref[1] + jnp.sum(x * x, axis=1)

        @pl.when(i == GRID - 1)
        def _():
            n = float(NCOMP)
            mean = acc_ref[0] / n
            var = acc_ref[1] / n - mean * mean
            sc = gam_ref[...] * lax.rsqrt(var + 1e-5)
            scale_ref[...] = sc
            shift_ref[...] = bet_ref[...] - mean * sc

    @pl.when(p == 1)
    def _():
        x = x_ref[...]
        x = x * scale_ref[...][:, None, :] + shift_ref[...][:, None, :]
        wi = wi_ref[...]
        wh = wh_ref[...]
        bi = bi_ref[...]
        bh = bh_ref[...]
        h = jnp.zeros((CBLK, HID), F32)
        for t in range(WIN):
            gi = jnp.dot(x[t], wi, preferred_element_type=F32) + bi
            gh = jnp.dot(h, wh, preferred_element_type=F32) + bh
            r = jax.nn.sigmoid(gi[:, :HID] + gh[:, :HID])
            z = jax.nn.sigmoid(gi[:, HID:2 * HID] + gh[:, HID:2 * HID])
            n = jnp.tanh(gi[:, 2 * HID:] + r * gh[:, 2 * HID:])
            h = (1.0 - z) * n + z * h
        seq_ref[...] = h
        h1 = jnp.dot(h, w1_ref[...], preferred_element_type=F32)
        aug_ref[0] = h1[:, :HSPLIT]
        aug_ref[1] = jnp.concatenate(
            [h1[:, HSPLIT:], jnp.ones((CBLK, HAF - (HID - HSPLIT)), F32)],
            axis=1)
        ss_ref[...] = jnp.dot(h1, asrc_ref[...], preferred_element_type=F32)
        sd_ref[...] = jnp.dot(h1, adst_ref[...], preferred_element_type=F32)


def _k1(daily, gam, bet, wi, wh, bi, bh, w1, asrc, adst):
    full = lambda shape: pl.BlockSpec(
        shape, lambda p, i: tuple(0 for _ in shape))
    return pl.pallas_call(
        _k1_body,
        grid=(2, GRID),
        in_specs=[
            pl.BlockSpec((WIN, CBLK, IND), lambda p, i: (0, i, 0)),
            full((WIN, IND)), full((WIN, IND)),
            full((IND, 3 * HID)), full((HID, 3 * HID)),
            full((1, 3 * HID)), full((1, 3 * HID)),
            full((HID, HID)), full((HID, 1)), full((HID, 1)),
        ],
        out_specs=[
            pl.BlockSpec((CBLK, HID), lambda p, i: (i, 0)),
            pl.BlockSpec((2, CBLK, HAF), lambda p, i: (0, i, 0)),
            pl.BlockSpec((CBLK, 1), lambda p, i: (i, 0)),
            pl.BlockSpec((CBLK, 1), lambda p, i: (i, 0)),
        ],
        out_shape=[
            jax.ShapeDtypeStruct((NCOMP, HID), F32),
            jax.ShapeDtypeStruct((2, NCOMP, HAF), F32),
            jax.ShapeDtypeStruct((NCOMP, 1), F32),
            jax.ShapeDtypeStruct((NCOMP, 1), F32),
        ],
        scratch_shapes=[
            pltpu.VMEM((2, WIN, IND), F32),
            pltpu.VMEM((WIN, IND), F32),
            pltpu.VMEM((WIN, IND), F32),
        ],
    )(daily, gam, bet, wi, wh, bi, bh, w1, asrc, adst)


NBUF = 5
RCHUNK = 50
SROUND = NCHUNK // RCHUNK


def _gat1_edges_body(src2_hbm, dst2_hbm, ssrc_hbm, sdst_hbm, aug2_hbm,
                     out_hbm, *refs):
    (ssrc_v, sdst_v, src2_v, dst2_v, exv_all, acc_sh) = refs[:6]
    bufs = refs[6:6 + NBUF]
    gsems = refs[6 + NBUF:6 + 2 * NBUF]
    ssems = refs[6 + 2 * NBUF:6 + 3 * NBUF]
    cid = lax.axis_index("c")
    sid = lax.axis_index("s")

    pltpu.sync_copy(ssrc_hbm, ssrc_v)
    pltpu.sync_copy(sdst_hbm, sdst_v)

    def z_body(i, carry):
        for j in range(HAF // 16):
            exv_all[i, pl.ds(j * 16, 16)] = jnp.zeros((16,), F32)
        return carry
    lax.fori_loop(0, RCHUNK, z_body, 0)
    row0 = sid * ROWS_T
    for z in range(ROWS_T // RCHUNK):
        pltpu.sync_copy(exv_all.at[pl.ds(0, RCHUNK)],
                        acc_sh.at[pl.ds(row0 + z * RCHUNK, RCHUNK)])
    pltpu.sync_copy(exv_all.at[pl.ds(0, ROWS_T % RCHUNK)],
                    acc_sh.at[pl.ds(row0 + ROWS_T - ROWS_T % RCHUNK,
                                    ROWS_T % RCHUNK)])
    plsc.subcore_barrier()

    rebase = cid * NCOMP

    def gather_start(b, kk):
        pltpu.async_copy(aug2_hbm.at[src2_v.at[kk]], bufs[b], gsems[b])

    def gather_wait(b, kk):
        pltpu.make_async_copy(aug2_hbm.at[src2_v.at[kk]], bufs[b],
                              gsems[b]).wait()

    def scat_start(b, kk):
        pltpu.async_copy(bufs[b], acc_sh.at[dst2_v.at[kk]], ssems[b],
                         add=True)

    def scat_wait(b, kk):
        pltpu.make_async_copy(bufs[b], acc_sh.at[dst2_v.at[kk]],
                              ssems[b]).wait()

    def weight(b, kk):
        def w_i(i, carry):
            exvec = exv_all[kk, pl.ds(i * 16, 16)]
            for l in range(16):
                a = exvec[l]
                row = i * 16 + l
                for j in range(HAF // 16):
                    sl = pl.ds(j * 16, 16)
                    bufs[b][row, sl] = bufs[b][row, sl] * a
            return carry
        lax.fori_loop(0, CHUNK // 16, w_i, 0)

    def super_round(r, carry):
        crow0 = sid * NCHUNK + r * RCHUNK
        pltpu.sync_copy(src2_hbm.at[pl.ds(crow0, RCHUNK)], src2_v)
        pltpu.sync_copy(dst2_hbm.at[pl.ds(crow0, RCHUNK)], dst2_v)

        def ex_body(k, c2):
            for i in range(CHUNK // 16):
                sl = pl.ds(i * 16, 16)
                s_ids = src2_v[k, sl]
                e = (plsc.load_gather(ssrc_v, [s_ids])
                     + plsc.load_gather(sdst_v, [dst2_v[k, sl]]))
                e = jnp.where(e >= 0.0, e, 0.2 * e)
                exv_all[k, sl] = jnp.exp(e)
                src2_v[k, sl] = s_ids + rebase
            return c2
        lax.fori_loop(0, RCHUNK, ex_body, 0)

        for b in range(NBUF):
            gather_start(b, b)

        def round_body(it, c2):
            k0 = it * NBUF
            for b in range(NBUF):
                kk = k0 + b
                gather_wait(b, kk)
                weight(b, kk)
                scat_start(b, kk)
            for b in range(NBUF):
                kk2 = k0 + b + NBUF

                @pl.when(kk2 < RCHUNK)
                def _():
                    scat_wait(b, k0 + b)
                    gather_start(b, kk2)
            return c2
        lax.fori_loop(0, RCHUNK // NBUF, round_body, 0)
        for b in range(NBUF):
            scat_wait(b, RCHUNK - NBUF + b)
        return carry
    lax.fori_loop(0, SROUND, super_round, 0)

    plsc.subcore_barrier()
    pltpu.sync_copy(acc_sh.at[pl.ds(row0, ROWS_T)],
                    out_hbm.at[pl.ds(cid * NCOMP + row0, ROWS_T)])


@functools.lru_cache(maxsize=1)
def _gat1_edges_call():
    mesh = plsc.VectorSubcoreMesh(
        core_axis_name="c", subcore_axis_name="s",
        num_cores=SC_CORES, num_subcores=SC_TILES)
    scratch = [
        pltpu.VMEM((NCOMP,), F32),
        pltpu.VMEM((NCOMP,), F32),
        pltpu.VMEM((RCHUNK, CHUNK), jnp.int32),
        pltpu.VMEM((RCHUNK, CHUNK), jnp.int32),
        pltpu.VMEM((RCHUNK, CHUNK), F32),
        pltpu.VMEM_SHARED((NCOMP, HAF), F32),
    ]
    scratch += [pltpu.VMEM((CHUNK, HAF), F32) for _ in range(NBUF)]
    scratch += [pltpu.SemaphoreType.DMA for _ in range(2 * NBUF)]
    return pl.kernel(
        _gat1_edges_body,
        out_type=jax.ShapeDtypeStruct((SC_CORES * NCOMP, HAF), F32),
        mesh=mesh,
        scratch_types=scratch,
        compiler_params=pltpu.CompilerParams(
            use_tc_tiling_on_sc=False, needs_layout_passes=False),
    )


def _gat1_edges(src, dst, ss, sd, aug2):
    src2 = src.reshape(NIN // CHUNK, CHUNK)
    dst2 = dst.reshape(NIN // CHUNK, CHUNK)
    return _gat1_edges_call()(src2, dst2, ss, sd, aug2)


def _k3_body(p0_ref, p1_ref, b1_ref, osrc_ref, odst_ref, w2_ref, a2s_ref,
             a2d_ref, b2_ref, seq_ref, fw_ref, fb_ref, lw_ref, lb_ref,
             out_ref, intra_s, sec_s, sec2_s):
    p = pl.program_id(0)
    i = pl.program_id(1)

    @pl.when(p == 0)
    def _():
        p0 = p0_ref[...]
        p1 = p1_ref[...]
        accv = jnp.concatenate([p0, p1[:, :HID - HSPLIT]], axis=1)
        den = p1[:, HID - HSPLIT:HID - HSPLIT + 1]
        intra = accv / (den + 1e-16) + b1_ref[...]
        intra_s[i] = intra
        sec_s[i] = jnp.max(intra.reshape(CBLK // PER, PER, HID), axis=1)

        @pl.when(i == GRID - 1)
        def _():
            sec = sec_s[...].reshape(NSEC, HID)
            h2 = jnp.dot(sec, w2_ref[...], preferred_element_type=F32)
            vs = jnp.dot(h2, a2s_ref[...], preferred_element_type=F32)
            vd = jnp.dot(h2, a2d_ref[...], preferred_element_type=F32)
            k = lax.broadcasted_iota(jnp.int32, (1, NSEC), 1)
            ohs = (osrc_ref[...] == k).astype(F32)
            ohd = (odst_ref[...] == k).astype(F32)
            e = (jnp.dot(ohs, vs, preferred_element_type=F32)
                 + jnp.dot(ohd, vd, preferred_element_type=F32))
            e = jnp.where(e >= 0.0, e, 0.2 * e)
            ex = jnp.exp(e)
            wdst = ohd * ex
            wss = lax.dot_general(wdst, ohs, (((0,), (0,)), ((), ())),
                                  preferred_element_type=F32)
            acc2 = jnp.dot(wss, h2, preferred_element_type=F32)
            den2 = jnp.sum(wss, axis=1, keepdims=True)
            sec2 = acc2 / (den2 + 1e-16) + b2_ref[...]
            sec2_s[...] = sec2.reshape(GRID, CBLK // PER, HID)

    @pl.when(p == 1)
    def _():
        secb = sec2_s[i]
        rep = jnp.broadcast_to(secb[:, None, :], (CBLK // PER, PER, HID))
        rep = rep.reshape(CBLK, HID)
        cat = jnp.concatenate([seq_ref[...], rep, intra_s[i]], axis=1)
        f = jnp.dot(cat, fw_ref[...], preferred_element_type=F32) + fb_ref[...]
        f = jnp.maximum(f, 0.0)
        lo = jnp.dot(f, lw_ref[...], preferred_element_type=F32) + lb_ref[...]
        m = jnp.max(lo, axis=1, keepdims=True)
        pe = jnp.exp(lo - m)
        sm = pe / jnp.sum(pe, axis=1, keepdims=True)
        ii = lax.broadcasted_iota(jnp.int32, (ODIM, ODIM), 0)
        jj = lax.broadcasted_iota(jnp.int32, (ODIM, ODIM), 1)
        tri = (ii <= jj).astype(F32)
        cum = jnp.dot(sm, tri, preferred_element_type=F32)
        out_ref[...] = jnp.clip(cum, 5e-8, 1.0 - 5e-8)


def _k3(parts, b1, osrc, odst, w2, a2s, a2d, b2, seq, fw, fb, lw, lb):
    full = lambda shape: pl.BlockSpec(
        shape, lambda p, i: tuple(0 for _ in shape))
    return pl.pallas_call(
        _k3_body,
        grid=(2, GRID),
        in_specs=[
            pl.BlockSpec((CBLK, HAF), lambda p, i: (i, 0)),
            pl.BlockSpec((CBLK, HAF), lambda p, i: (i + GRID, 0)),
            full((1, HID)),
            full((NOUT, 1)), full((NOUT, 1)),
            full((HID, HID)), full((HID, 1)), full((HID, 1)), full((1, HID)),
            pl.BlockSpec((CBLK, HID), lambda p, i: (i, 0)),
            full((3 * HID, HID)), full((1, HID)),
            full((HID, ODIM)), full((1, ODIM)),
        ],
        out_specs=[pl.BlockSpec((CBLK, ODIM), lambda p, i: (i, 0))],
        out_shape=[jax.ShapeDtypeStruct((NCOMP, ODIM), F32)],
        scratch_shapes=[
            pltpu.VMEM((GRID, CBLK, HID), F32),
            pltpu.VMEM((GRID, CBLK // PER, HID), F32),
            pltpu.VMEM((GRID, CBLK // PER, HID), F32),
        ],
    )(parts, parts, b1, osrc, odst, w2, a2s, a2d, b2, seq, fw, fb, lw, lb)[0]


def kernel(daily_data_batch, inner_edge, outer_edge, bn_gamma, bn_beta,
           gru_Wi, gru_Wh, gru_bi, gru_bh, gat1_W, gat1_a_src, gat1_a_dst,
           gat1_b, gat2_W, gat2_a_src, gat2_a_dst, gat2_b, fusion_W,
           fusion_b, logit_W, logit_b):
    gam = bn_gamma.reshape(WIN, IND)
    bet = bn_beta.reshape(WIN, IND)
    seq, aug3, ss, sd = _k1(
        daily_data_batch, gam, bet, gru_Wi, gru_Wh,
        gru_bi.reshape(1, 3 * HID), gru_bh.reshape(1, 3 * HID),
        gat1_W, gat1_a_src.reshape(HID, 1), gat1_a_dst.reshape(HID, 1))
    parts = _gat1_edges(inner_edge[0], inner_edge[1],
                        ss.reshape(NCOMP), sd.reshape(NCOMP),
                        aug3.reshape(SC_CORES * NCOMP, HAF))
    return _k3(parts, gat1_b.reshape(1, HID),
               outer_edge[0].reshape(NOUT, 1), outer_edge[1].reshape(NOUT, 1),
               gat2_W, gat2_a_src.reshape(HID, 1), gat2_a_dst.reshape(HID, 1),
               gat2_b.reshape(1, HID), seq, fusion_W,
               fusion_b.reshape(1, HID), logit_W, logit_b.reshape(1, ODIM))
